# baseline (speedup 1.0000x reference)
import functools

import jax
import jax.numpy as jnp
from jax import lax
from jax.experimental import pallas as pl
from jax.experimental.pallas import tpu as pltpu
from jax.experimental.pallas import tpu_sc as plsc

_B, _H, _N, _D = 1, 16, 2048, 64
_K = 32
_QBLK = 128
_NEG = -1e30
_L = 16
_NCHUNK = _N // _L
_NW = 32


def _rot(v, sh):
    idx = ((lax.iota(jnp.int32, _L) + sh) & (_L - 1))[:, None]
    return lax.gather(
        v,
        idx,
        lax.GatherDimensionNumbers(
            offset_dims=(), collapsed_slice_dims=(0,), start_index_map=(0,)
        ),
        slice_sizes=(1,),
        mode=lax.GatherScatterMode.PROMISE_IN_BOUNDS,
    )


def _vmax_all(v):
    for sh in (8, 4, 2, 1):
        v = jnp.maximum(v, _rot(v, sh))
    return v


def _vmin_all(v):
    for sh in (8, 4, 2, 1):
        v = jnp.minimum(v, _rot(v, sh))
    return v


def _sc_topk_body(rows_per_w, mask_hbm, bias_hbm, xrow, brow, cm):
    wid = lax.axis_index("s") * 2 + lax.axis_index("c")
    base = wid * rows_per_w
    lane_iota = lax.iota(jnp.int32, _L)

    def row_loop(r, _):
        row = base + r
        pltpu.sync_copy(mask_hbm.at[row], xrow)

        def _cm_write(c, val):
            vbase = (c // _L) * _L
            cmv = cm[pl.ds(vbase, _L)]
            cm[pl.ds(vbase, _L)] = jnp.where(lane_iota == c % _L, val, cmv)

        def cm_init(c, _):
            _cm_write(c, _vmax_all(xrow[pl.ds(c * _L, _L)]))
            return 0

        lax.fori_loop(0, _NCHUNK, cm_init, 0)

        def round_fn(t, _):
            def tree(j, carry):
                v0, i0 = carry
                v1 = cm[pl.ds(j * _L, _L)]
                i1 = lane_iota + j * _L
                take = v1 > v0
                return jnp.where(take, v1, v0), jnp.where(take, i1, i0)

            vv, vi = lax.fori_loop(
                1, _NCHUNK // _L, tree, (cm[pl.ds(0, _L)], lane_iota)
            )
            m = _vmax_all(vv)
            c = _vmin_all(jnp.where(vv >= m, vi, _N))[0]
            v = xrow[pl.ds(c * _L, _L)]
            l0 = _vmin_all(jnp.where(v >= m, lane_iota, _L))
            vnew = jnp.where(lane_iota == l0, _NEG, v)
            xrow[pl.ds(c * _L, _L)] = vnew
            _cm_write(c, _vmax_all(vnew))
            return 0

        lax.fori_loop(0, _K, round_fn, 0)

        def bias_fn(cc, _):
            v = xrow[pl.ds(cc * _L, _L)]
            brow[pl.ds(cc * _L, _L)] = jnp.where(v < 0.0, 0.0, _NEG)
            return 0

        lax.fori_loop(0, _NCHUNK, bias_fn, 0)
        pltpu.sync_copy(brow, bias_hbm.at[row])
        return 0

    lax.fori_loop(0, rows_per_w, row_loop, 0)


def _sc_topk_bias(mask2d):
    nrows = mask2d.shape[0]
    mesh = plsc.VectorSubcoreMesh(core_axis_name="c", subcore_axis_name="s")
    fn = pl.kernel(
        functools.partial(_sc_topk_body, nrows // _NW),
        out_type=jax.ShapeDtypeStruct((nrows, _N), jnp.float32),
        mesh=mesh,
        scratch_types=[
            pltpu.VMEM((_N,), jnp.float32),
            pltpu.VMEM((_N,), jnp.float32),
            pltpu.VMEM((_NCHUNK,), jnp.float32),
        ],
    )
    return fn(mask2d)


def _attn_body(bias_ref, q_ref, k_ref, v_ref, o_ref):
    bias = bias_ref[...]
    for h in range(_H):
        q = q_ref[0, h] * 0.125
        k = k_ref[0, h]
        v = v_ref[0, h]
        s = jax.lax.dot_general(
            q, k, (((1,), (1,)), ((), ())), preferred_element_type=jnp.float32
        )
        e = jnp.exp(s + bias)
        r = 1.0 / jnp.sum(e, axis=1, keepdims=True)
        o = jax.lax.dot_general(
            e, v, (((1,), (0,)), ((), ())), preferred_element_type=jnp.float32
        )
        o_ref[0, h] = o * r


def _tc_attention(query, key, value, bias, row0, nrows):
    grid = (nrows // _QBLK,)
    blk0 = row0 // _QBLK
    return pl.pallas_call(
        _attn_body,
        grid=grid,
        in_specs=[
            pl.BlockSpec((_QBLK, _N), lambda i: (i, 0)),
            pl.BlockSpec((1, _H, _QBLK, _D), lambda i, b=blk0: (0, 0, b + i, 0)),
            pl.BlockSpec((1, _H, _N, _D), lambda i: (0, 0, 0, 0)),
            pl.BlockSpec((1, _H, _N, _D), lambda i: (0, 0, 0, 0)),
        ],
        out_specs=pl.BlockSpec((1, _H, _QBLK, _D), lambda i: (0, 0, i, 0)),
        out_shape=jax.ShapeDtypeStruct((1, _H, nrows, _D), jnp.float32),
        compiler_params=pltpu.CompilerParams(
            dimension_semantics=("arbitrary",),
        ),
    )(bias, query, key, value)


@jax.jit
def kernel(query, key, value, attention_mask):
    mask2d = attention_mask[0]
    half = _N // 2
    bias_top = _sc_topk_bias(mask2d[:half])
    bias_bot = _sc_topk_bias(mask2d[half:])
    out_top = _tc_attention(query, key, value, bias_top, 0, half)
    out_bot = _tc_attention(query, key, value, bias_bot, half, half)
    return jnp.concatenate([out_top, out_bot], axis=2)


# speedup vs baseline: 1.3429x; 1.3429x over previous
import functools

import jax
import jax.numpy as jnp
from jax import lax
from jax.experimental import pallas as pl
from jax.experimental.pallas import tpu as pltpu
from jax.experimental.pallas import tpu_sc as plsc

_B, _H, _N, _D = 1, 16, 2048, 64
_K = 32
_QBLK = 128
_NEG = -1e30
_L = 16
_NCHUNK = _N // _L
_NW = 32


def _rot(v, sh):
    idx = ((lax.iota(jnp.int32, _L) + sh) & (_L - 1))[:, None]
    return lax.gather(
        v,
        idx,
        lax.GatherDimensionNumbers(
            offset_dims=(), collapsed_slice_dims=(0,), start_index_map=(0,)
        ),
        slice_sizes=(1,),
        mode=lax.GatherScatterMode.PROMISE_IN_BOUNDS,
    )


def _vmax_all(v):
    for sh in (8, 4, 2, 1):
        v = jnp.maximum(v, _rot(v, sh))
    return v


def _vmin_all(v):
    for sh in (8, 4, 2, 1):
        v = jnp.minimum(v, _rot(v, sh))
    return v


_RG = 4


def _sc_topk_body(rows_per_w, mask_hbm, bias_hbm, xrow, brow, cm):
    wid = lax.axis_index("s") * 2 + lax.axis_index("c")
    base = wid * rows_per_w
    lane_iota = lax.iota(jnp.int32, _L)

    def _cm_write(i, c, val):
        vbase = (c // _L) * _L
        cmv = cm[i, pl.ds(vbase, _L)]
        cm[i, pl.ds(vbase, _L)] = jnp.where(lane_iota == c % _L, val, cmv)

    def row_loop(r, _):
        row = base + r * _RG
        pltpu.sync_copy(mask_hbm.at[pl.ds(row, _RG)], xrow)

        def cm_init(c, _):
            for i in range(_RG):
                _cm_write(i, c, _vmax_all(xrow[i, pl.ds(c * _L, _L)]))
            return 0

        lax.fori_loop(0, _NCHUNK, cm_init, 0)

        def round_fn(t, _):
            def tree(j, carry):
                out = []
                for i in range(_RG):
                    v0, i0 = carry[i]
                    v1 = cm[i, pl.ds(j * _L, _L)]
                    i1 = lane_iota + j * _L
                    take = v1 > v0
                    out.append(
                        (jnp.where(take, v1, v0), jnp.where(take, i1, i0))
                    )
                return tuple(out)

            init = tuple(
                (cm[i, pl.ds(0, _L)], lane_iota) for i in range(_RG)
            )
            red = lax.fori_loop(1, _NCHUNK // _L, tree, init)
            for i in range(_RG):
                vv, vi = red[i]
                m = _vmax_all(vv)
                c = _vmin_all(jnp.where(vv >= m, vi, _N))[0]
                v = xrow[i, pl.ds(c * _L, _L)]
                l0 = _vmin_all(jnp.where(v >= m, lane_iota, _L))
                vnew = jnp.where(lane_iota == l0, _NEG, v)
                xrow[i, pl.ds(c * _L, _L)] = vnew
                _cm_write(i, c, _vmax_all(vnew))
            return 0

        lax.fori_loop(0, _K, round_fn, 0)

        def bias_fn(cc, _):
            for i in range(_RG):
                v = xrow[i, pl.ds(cc * _L, _L)]
                brow[i, pl.ds(cc * _L, _L)] = jnp.where(v < 0.0, 0.0, _NEG)
            return 0

        lax.fori_loop(0, _NCHUNK, bias_fn, 0)
        pltpu.sync_copy(brow, bias_hbm.at[pl.ds(row, _RG)])
        return 0

    lax.fori_loop(0, rows_per_w // _RG, row_loop, 0)


def _sc_topk_bias(mask2d):
    nrows = mask2d.shape[0]
    mesh = plsc.VectorSubcoreMesh(core_axis_name="c", subcore_axis_name="s")
    fn = pl.kernel(
        functools.partial(_sc_topk_body, nrows // _NW),
        out_type=jax.ShapeDtypeStruct((nrows, _N), jnp.float32),
        mesh=mesh,
        scratch_types=[
            pltpu.VMEM((_RG, _N), jnp.float32),
            pltpu.VMEM((_RG, _N), jnp.float32),
            pltpu.VMEM((_RG, _NCHUNK), jnp.float32),
        ],
    )
    return fn(mask2d)


def _attn_body(bias_ref, q_ref, k_ref, v_ref, o_ref):
    bias = bias_ref[...]
    for h in range(_H):
        q = q_ref[0, h] * 0.125
        k = k_ref[0, h]
        v = v_ref[0, h]
        s = jax.lax.dot_general(
            q, k, (((1,), (1,)), ((), ())), preferred_element_type=jnp.float32
        )
        e = jnp.exp(s + bias)
        r = 1.0 / jnp.sum(e, axis=1, keepdims=True)
        o = jax.lax.dot_general(
            e, v, (((1,), (0,)), ((), ())), preferred_element_type=jnp.float32
        )
        o_ref[0, h] = o * r


def _tc_attention(query, key, value, bias, row0, nrows):
    grid = (nrows // _QBLK,)
    blk0 = row0 // _QBLK
    return pl.pallas_call(
        _attn_body,
        grid=grid,
        in_specs=[
            pl.BlockSpec((_QBLK, _N), lambda i: (i, 0)),
            pl.BlockSpec((1, _H, _QBLK, _D), lambda i, b=blk0: (0, 0, b + i, 0)),
            pl.BlockSpec((1, _H, _N, _D), lambda i: (0, 0, 0, 0)),
            pl.BlockSpec((1, _H, _N, _D), lambda i: (0, 0, 0, 0)),
        ],
        out_specs=pl.BlockSpec((1, _H, _QBLK, _D), lambda i: (0, 0, i, 0)),
        out_shape=jax.ShapeDtypeStruct((1, _H, nrows, _D), jnp.float32),
        compiler_params=pltpu.CompilerParams(
            dimension_semantics=("arbitrary",),
        ),
    )(bias, query, key, value)


@jax.jit
def kernel(query, key, value, attention_mask):
    mask2d = attention_mask[0]
    half = _N // 2
    bias_top = _sc_topk_bias(mask2d[:half])
    bias_bot = _sc_topk_bias(mask2d[half:])
    out_top = _tc_attention(query, key, value, bias_top, 0, half)
    out_bot = _tc_attention(query, key, value, bias_bot, half, half)
    return jnp.concatenate([out_top, out_bot], axis=2)


# speedup vs baseline: 1.3974x; 1.0406x over previous
import functools

import jax
import jax.numpy as jnp
from jax import lax
from jax.experimental import pallas as pl
from jax.experimental.pallas import tpu as pltpu
from jax.experimental.pallas import tpu_sc as plsc

_B, _H, _N, _D = 1, 16, 2048, 64
_K = 32
_QBLK = 128
_NEG = -1e30
_L = 16
_NCHUNK = _N // _L
_NW = 32


def _rot(v, sh):
    idx = ((lax.iota(jnp.int32, _L) + sh) & (_L - 1))[:, None]
    return lax.gather(
        v,
        idx,
        lax.GatherDimensionNumbers(
            offset_dims=(), collapsed_slice_dims=(0,), start_index_map=(0,)
        ),
        slice_sizes=(1,),
        mode=lax.GatherScatterMode.PROMISE_IN_BOUNDS,
    )


def _vmax_all(v):
    for sh in (8, 4, 2, 1):
        v = jnp.maximum(v, _rot(v, sh))
    return v


def _vmin_all(v):
    for sh in (8, 4, 2, 1):
        v = jnp.minimum(v, _rot(v, sh))
    return v


_RG = 8


def _sc_topk_body(rows_per_w, mask_hbm, bias_hbm, xrow, brow, cm):
    wid = lax.axis_index("s") * 2 + lax.axis_index("c")
    base = wid * rows_per_w
    lane_iota = lax.iota(jnp.int32, _L)

    def _cm_write(i, c, val):
        vbase = (c // _L) * _L
        cmv = cm[i, pl.ds(vbase, _L)]
        cm[i, pl.ds(vbase, _L)] = jnp.where(lane_iota == c % _L, val, cmv)

    def row_loop(r, _):
        row = base + r * _RG
        pltpu.sync_copy(mask_hbm.at[pl.ds(row, _RG)], xrow)

        def cm_init(c, _):
            for i in range(_RG):
                _cm_write(i, c, _vmax_all(xrow[i, pl.ds(c * _L, _L)]))
            return 0

        lax.fori_loop(0, _NCHUNK, cm_init, 0)

        def round_fn(t, _):
            def tree(j, carry):
                out = []
                for i in range(_RG):
                    v0, i0 = carry[i]
                    v1 = cm[i, pl.ds(j * _L, _L)]
                    i1 = lane_iota + j * _L
                    take = v1 > v0
                    out.append(
                        (jnp.where(take, v1, v0), jnp.where(take, i1, i0))
                    )
                return tuple(out)

            init = tuple(
                (cm[i, pl.ds(0, _L)], lane_iota) for i in range(_RG)
            )
            red = lax.fori_loop(1, _NCHUNK // _L, tree, init)
            for i in range(_RG):
                vv, vi = red[i]
                m = _vmax_all(vv)
                c = _vmin_all(jnp.where(vv >= m, vi, _N))[0]
                v = xrow[i, pl.ds(c * _L, _L)]
                l0 = _vmin_all(jnp.where(v >= m, lane_iota, _L))
                vnew = jnp.where(lane_iota == l0, _NEG, v)
                xrow[i, pl.ds(c * _L, _L)] = vnew
                _cm_write(i, c, _vmax_all(vnew))
            return 0

        lax.fori_loop(0, _K, round_fn, 0)

        def bias_fn(cc, _):
            for i in range(_RG):
                v = xrow[i, pl.ds(cc * _L, _L)]
                brow[i, pl.ds(cc * _L, _L)] = jnp.where(v < 0.0, 0.0, _NEG)
            return 0

        lax.fori_loop(0, _NCHUNK, bias_fn, 0)
        pltpu.sync_copy(brow, bias_hbm.at[pl.ds(row, _RG)])
        return 0

    lax.fori_loop(0, rows_per_w // _RG, row_loop, 0)


def _sc_topk_bias(mask2d):
    nrows = mask2d.shape[0]
    mesh = plsc.VectorSubcoreMesh(core_axis_name="c", subcore_axis_name="s")
    fn = pl.kernel(
        functools.partial(_sc_topk_body, nrows // _NW),
        out_type=jax.ShapeDtypeStruct((nrows, _N), jnp.float32),
        mesh=mesh,
        scratch_types=[
            pltpu.VMEM((_RG, _N), jnp.float32),
            pltpu.VMEM((_RG, _N), jnp.float32),
            pltpu.VMEM((_RG, _NCHUNK), jnp.float32),
        ],
    )
    return fn(mask2d)


def _attn_body(bias_ref, q_ref, k_ref, v_ref, o_ref):
    bias = bias_ref[...]
    for h in range(_H):
        q = q_ref[0, h] * 0.125
        k = k_ref[0, h]
        v = v_ref[0, h]
        s = jax.lax.dot_general(
            q, k, (((1,), (1,)), ((), ())), preferred_element_type=jnp.float32
        )
        e = jnp.exp(s + bias)
        r = 1.0 / jnp.sum(e, axis=1, keepdims=True)
        o = jax.lax.dot_general(
            e, v, (((1,), (0,)), ((), ())), preferred_element_type=jnp.float32
        )
        o_ref[0, h] = o * r


def _tc_attention(query, key, value, bias, row0, nrows):
    grid = (nrows // _QBLK,)
    blk0 = row0 // _QBLK
    return pl.pallas_call(
        _attn_body,
        grid=grid,
        in_specs=[
            pl.BlockSpec((_QBLK, _N), lambda i: (i, 0)),
            pl.BlockSpec((1, _H, _QBLK, _D), lambda i, b=blk0: (0, 0, b + i, 0)),
            pl.BlockSpec((1, _H, _N, _D), lambda i: (0, 0, 0, 0)),
            pl.BlockSpec((1, _H, _N, _D), lambda i: (0, 0, 0, 0)),
        ],
        out_specs=pl.BlockSpec((1, _H, _QBLK, _D), lambda i: (0, 0, i, 0)),
        out_shape=jax.ShapeDtypeStruct((1, _H, nrows, _D), jnp.float32),
        compiler_params=pltpu.CompilerParams(
            dimension_semantics=("arbitrary",),
        ),
    )(bias, query, key, value)


@jax.jit
def kernel(query, key, value, attention_mask):
    mask2d = attention_mask[0]
    half = _N // 2
    bias_top = _sc_topk_bias(mask2d[:half])
    bias_bot = _sc_topk_bias(mask2d[half:])
    out_top = _tc_attention(query, key, value, bias_top, 0, half)
    out_bot = _tc_attention(query, key, value, bias_bot, half, half)
    return jnp.concatenate([out_top, out_bot], axis=2)


# speedup vs baseline: 1.4279x; 1.0218x over previous
import functools

import jax
import jax.numpy as jnp
from jax import lax
from jax.experimental import pallas as pl
from jax.experimental.pallas import tpu as pltpu
from jax.experimental.pallas import tpu_sc as plsc

_B, _H, _N, _D = 1, 16, 2048, 64
_K = 32
_QBLK = 128
_NEG = -1e30
_L = 16
_NCHUNK = _N // _L
_NW = 32


def _rot(v, sh):
    idx = ((lax.iota(jnp.int32, _L) + sh) & (_L - 1))[:, None]
    return lax.gather(
        v,
        idx,
        lax.GatherDimensionNumbers(
            offset_dims=(), collapsed_slice_dims=(0,), start_index_map=(0,)
        ),
        slice_sizes=(1,),
        mode=lax.GatherScatterMode.PROMISE_IN_BOUNDS,
    )


def _vmax_all(v):
    for sh in (8, 4, 2, 1):
        v = jnp.maximum(v, _rot(v, sh))
    return v


def _vmin_all(v):
    for sh in (8, 4, 2, 1):
        v = jnp.minimum(v, _rot(v, sh))
    return v


_RG = 8


def _sc_topk_body(rows_per_w, mask_hbm, bias_hbm, xrow, brow, cm):
    wid = lax.axis_index("s") * 2 + lax.axis_index("c")
    base = wid * rows_per_w
    lane_iota = lax.iota(jnp.int32, _L)

    def _cm_write(i, c, val):
        vbase = (c // _L) * _L
        cmv = cm[i, pl.ds(vbase, _L)]
        cm[i, pl.ds(vbase, _L)] = jnp.where(lane_iota == c % _L, val, cmv)

    def row_loop(r, _):
        row = base + r * _RG
        pltpu.sync_copy(mask_hbm.at[pl.ds(row, _RG)], xrow)

        def cm_init(c, _):
            for i in range(_RG):
                _cm_write(i, c, _vmax_all(xrow[i, pl.ds(c * _L, _L)]))
            return 0

        lax.fori_loop(0, _NCHUNK, cm_init, 0)

        def round_fn(t, _):
            def tree(j, carry):
                out = []
                for i in range(_RG):
                    v0, i0 = carry[i]
                    v1 = cm[i, pl.ds(j * _L, _L)]
                    i1 = lane_iota + j * _L
                    take = v1 > v0
                    out.append(
                        (jnp.where(take, v1, v0), jnp.where(take, i1, i0))
                    )
                return tuple(out)

            init = tuple(
                (cm[i, pl.ds(0, _L)], lane_iota) for i in range(_RG)
            )
            red = lax.fori_loop(1, _NCHUNK // _L, tree, init)
            for i in range(_RG):
                vv, vi = red[i]
                m = _vmax_all(vv)
                c = _vmin_all(jnp.where(vv >= m, vi, _N))[0]
                v = xrow[i, pl.ds(c * _L, _L)]
                l0 = _vmin_all(jnp.where(v >= m, lane_iota, _L))
                vnew = jnp.where(lane_iota == l0, _NEG, v)
                xrow[i, pl.ds(c * _L, _L)] = vnew
                _cm_write(i, c, _vmax_all(vnew))
            return 0

        lax.fori_loop(0, _K, round_fn, 0)

        def bias_fn(cc, _):
            for i in range(_RG):
                v = xrow[i, pl.ds(cc * _L, _L)]
                brow[i, pl.ds(cc * _L, _L)] = jnp.where(v < 0.0, 0.0, _NEG)
            return 0

        lax.fori_loop(0, _NCHUNK, bias_fn, 0)
        pltpu.sync_copy(brow, bias_hbm.at[pl.ds(row, _RG)])
        return 0

    lax.fori_loop(0, rows_per_w // _RG, row_loop, 0)


def _sc_topk_bias(mask2d):
    nrows = mask2d.shape[0]
    mesh = plsc.VectorSubcoreMesh(core_axis_name="c", subcore_axis_name="s")
    fn = pl.kernel(
        functools.partial(_sc_topk_body, nrows // _NW),
        out_type=jax.ShapeDtypeStruct((nrows, _N), jnp.float32),
        mesh=mesh,
        scratch_types=[
            pltpu.VMEM((_RG, _N), jnp.float32),
            pltpu.VMEM((_RG, _N), jnp.float32),
            pltpu.VMEM((_RG, _NCHUNK), jnp.float32),
        ],
    )
    return fn(mask2d)


def _fused_body(mask_ref, q_ref, k_ref, v_ref, o_ref):
    x = mask_ref[0]
    iota = jax.lax.broadcasted_iota(jnp.int32, (_QBLK, _N), 1)

    def step(_, x):
        m = jnp.max(x, axis=1, keepdims=True)
        fi = jnp.min(jnp.where(x >= m, iota, _N), axis=1, keepdims=True)
        return jnp.where(iota == fi, _NEG, x)

    x = jax.lax.fori_loop(0, _K, step, x, unroll=True)
    bias = jnp.where(x < 0.0, 0.0, _NEG)
    _heads(bias, q_ref, k_ref, v_ref, o_ref)


def _heads(bias, q_ref, k_ref, v_ref, o_ref):
    for h in range(_H):
        q = q_ref[0, h] * 0.125
        k = k_ref[0, h]
        v = v_ref[0, h]
        s = jax.lax.dot_general(
            q, k, (((1,), (1,)), ((), ())), preferred_element_type=jnp.float32
        )
        e = jnp.exp(s + bias)
        r = 1.0 / jnp.sum(e, axis=1, keepdims=True)
        o = jax.lax.dot_general(
            e, v, (((1,), (0,)), ((), ())), preferred_element_type=jnp.float32
        )
        o_ref[0, h] = o * r


def _tc_fused(query, key, value, mask, nrows):
    grid = (nrows // _QBLK,)
    return pl.pallas_call(
        _fused_body,
        grid=grid,
        in_specs=[
            pl.BlockSpec((1, _QBLK, _N), lambda i: (0, i, 0)),
            pl.BlockSpec((1, _H, _QBLK, _D), lambda i: (0, 0, i, 0)),
            pl.BlockSpec((1, _H, _N, _D), lambda i: (0, 0, 0, 0)),
            pl.BlockSpec((1, _H, _N, _D), lambda i: (0, 0, 0, 0)),
        ],
        out_specs=pl.BlockSpec((1, _H, _QBLK, _D), lambda i: (0, 0, i, 0)),
        out_shape=jax.ShapeDtypeStruct((1, _H, nrows, _D), jnp.float32),
        compiler_params=pltpu.CompilerParams(
            dimension_semantics=("arbitrary",),
        ),
    )(mask, query, key, value)


def _attn_body(bias_ref, q_ref, k_ref, v_ref, o_ref):
    _heads(bias_ref[...], q_ref, k_ref, v_ref, o_ref)


def _tc_attention(query, key, value, bias, row0, nrows):
    grid = (nrows // _QBLK,)
    blk0 = row0 // _QBLK
    return pl.pallas_call(
        _attn_body,
        grid=grid,
        in_specs=[
            pl.BlockSpec((_QBLK, _N), lambda i: (i, 0)),
            pl.BlockSpec((1, _H, _QBLK, _D), lambda i, b=blk0: (0, 0, b + i, 0)),
            pl.BlockSpec((1, _H, _N, _D), lambda i: (0, 0, 0, 0)),
            pl.BlockSpec((1, _H, _N, _D), lambda i: (0, 0, 0, 0)),
        ],
        out_specs=pl.BlockSpec((1, _H, _QBLK, _D), lambda i: (0, 0, i, 0)),
        out_shape=jax.ShapeDtypeStruct((1, _H, nrows, _D), jnp.float32),
        compiler_params=pltpu.CompilerParams(
            dimension_semantics=("arbitrary",),
        ),
    )(bias, query, key, value)


@jax.jit
def kernel(query, key, value, attention_mask):
    mask2d = attention_mask[0]
    half = _N // 2
    bias_bot = _sc_topk_bias(mask2d[half:])
    out_top = _tc_fused(query, key, value, attention_mask[:, :half], half)
    out_bot = _tc_attention(query, key, value, bias_bot, half, half)
    return jnp.concatenate([out_top, out_bot], axis=2)


# speedup vs baseline: 1.6435x; 1.1509x over previous
import functools

import jax
import jax.numpy as jnp
from jax import lax
from jax.experimental import pallas as pl
from jax.experimental.pallas import tpu as pltpu
from jax.experimental.pallas import tpu_sc as plsc

_B, _H, _N, _D = 1, 16, 2048, 64
_K = 32
_QBLK = 128
_NEG = -1e30
_L = 16
_NCHUNK = _N // _L
_NW = 32


def _rot(v, sh):
    idx = ((lax.iota(jnp.int32, _L) + sh) & (_L - 1))[:, None]
    return lax.gather(
        v,
        idx,
        lax.GatherDimensionNumbers(
            offset_dims=(), collapsed_slice_dims=(0,), start_index_map=(0,)
        ),
        slice_sizes=(1,),
        mode=lax.GatherScatterMode.PROMISE_IN_BOUNDS,
    )


def _vmax_all(v):
    for sh in (8, 4, 2, 1):
        v = jnp.maximum(v, _rot(v, sh))
    return v


def _vmin_all(v):
    for sh in (8, 4, 2, 1):
        v = jnp.minimum(v, _rot(v, sh))
    return v


_TOP_ROWS = 768
_RG = 8


def _sc_topk_body(rows_per_w, row0, mask_hbm, bias_hbm, xrow, brow, cm):
    wid = lax.axis_index("s") * 2 + lax.axis_index("c")
    base = wid * rows_per_w
    lane_iota = lax.iota(jnp.int32, _L)

    def _cm_write(i, c, val):
        vbase = (c // _L) * _L
        cmv = cm[i, pl.ds(vbase, _L)]
        cm[i, pl.ds(vbase, _L)] = jnp.where(lane_iota == c % _L, val, cmv)

    def row_loop(r, _):
        row = base + r * _RG
        pltpu.sync_copy(mask_hbm.at[pl.ds(row0 + row, _RG)], xrow)

        def cm_init(c, _):
            for i in range(_RG):
                _cm_write(i, c, _vmax_all(xrow[i, pl.ds(c * _L, _L)]))
            return 0

        lax.fori_loop(0, _NCHUNK, cm_init, 0)

        def round_fn(t, _):
            def tree(j, carry):
                out = []
                for i in range(_RG):
                    v0, i0 = carry[i]
                    v1 = cm[i, pl.ds(j * _L, _L)]
                    i1 = lane_iota + j * _L
                    take = v1 > v0
                    out.append(
                        (jnp.where(take, v1, v0), jnp.where(take, i1, i0))
                    )
                return tuple(out)

            init = tuple(
                (cm[i, pl.ds(0, _L)], lane_iota) for i in range(_RG)
            )
            red = lax.fori_loop(1, _NCHUNK // _L, tree, init)
            for i in range(_RG):
                vv, vi = red[i]
                m = _vmax_all(vv)
                c = _vmin_all(jnp.where(vv >= m, vi, _N))[0]
                v = xrow[i, pl.ds(c * _L, _L)]
                l0 = _vmin_all(jnp.where(v >= m, lane_iota, _L))
                vnew = jnp.where(lane_iota == l0, _NEG, v)
                xrow[i, pl.ds(c * _L, _L)] = vnew
                _cm_write(i, c, _vmax_all(vnew))
            return 0

        lax.fori_loop(0, _K, round_fn, 0)

        def bias_fn(cc, _):
            for i in range(_RG):
                v = xrow[i, pl.ds(cc * _L, _L)]
                brow[i, pl.ds(cc * _L, _L)] = jnp.where(v < 0.0, 0.0, _NEG)
            return 0

        lax.fori_loop(0, _NCHUNK, bias_fn, 0)
        pltpu.sync_copy(brow, bias_hbm.at[pl.ds(row, _RG)])
        return 0

    lax.fori_loop(0, rows_per_w // _RG, row_loop, 0)


def _sc_topk_bias(mask2d, row0, nrows):
    mesh = plsc.VectorSubcoreMesh(core_axis_name="c", subcore_axis_name="s")
    fn = pl.kernel(
        functools.partial(_sc_topk_body, nrows // _NW, row0),
        out_type=jax.ShapeDtypeStruct((nrows, _N), jnp.float32),
        mesh=mesh,
        scratch_types=[
            pltpu.VMEM((_RG, _N), jnp.float32),
            pltpu.VMEM((_RG, _N), jnp.float32),
            pltpu.VMEM((_RG, _NCHUNK), jnp.float32),
        ],
    )
    return fn(mask2d)


def _fused_body(mask_ref, q_ref, k_ref, v_ref, o_ref):
    x = mask_ref[0]
    iota = jax.lax.broadcasted_iota(jnp.int32, (_QBLK, _N), 1)

    def step(_, x):
        m = jnp.max(x, axis=1, keepdims=True)
        fi = jnp.min(jnp.where(x >= m, iota, _N), axis=1, keepdims=True)
        return jnp.where(iota == fi, _NEG, x)

    x = jax.lax.fori_loop(0, _K, step, x, unroll=True)
    bias = jnp.where(x < 0.0, 0.0, _NEG)
    _heads(bias, q_ref, k_ref, v_ref, o_ref)


def _heads(bias, q_ref, k_ref, v_ref, o_ref):
    for h in range(_H):
        q = q_ref[0, h] * 0.125
        k = k_ref[0, h]
        v = v_ref[0, h]
        s = jax.lax.dot_general(
            q, k, (((1,), (1,)), ((), ())), preferred_element_type=jnp.float32
        )
        e = jnp.exp(s + bias)
        r = 1.0 / jnp.sum(e, axis=1, keepdims=True)
        o = jax.lax.dot_general(
            e, v, (((1,), (0,)), ((), ())), preferred_element_type=jnp.float32
        )
        o_ref[0, h] = o * r


def _tc_fused(query, key, value, mask, nrows):
    grid = (nrows // _QBLK,)
    return pl.pallas_call(
        _fused_body,
        grid=grid,
        in_specs=[
            pl.BlockSpec((1, _QBLK, _N), lambda i: (0, i, 0)),
            pl.BlockSpec((1, _H, _QBLK, _D), lambda i: (0, 0, i, 0)),
            pl.BlockSpec((1, _H, _N, _D), lambda i: (0, 0, 0, 0)),
            pl.BlockSpec((1, _H, _N, _D), lambda i: (0, 0, 0, 0)),
        ],
        out_specs=pl.BlockSpec((1, _H, _QBLK, _D), lambda i: (0, 0, i, 0)),
        out_shape=jax.ShapeDtypeStruct((1, _H, nrows, _D), jnp.float32),
        compiler_params=pltpu.CompilerParams(
            dimension_semantics=("arbitrary",),
        ),
    )(mask, query, key, value)


def _attn_body(bias_ref, q_ref, k_ref, v_ref, o_ref):
    _heads(bias_ref[...], q_ref, k_ref, v_ref, o_ref)


def _tc_attention(query, key, value, bias, row0, nrows):
    grid = (nrows // _QBLK,)
    blk0 = row0 // _QBLK
    return pl.pallas_call(
        _attn_body,
        grid=grid,
        in_specs=[
            pl.BlockSpec((_QBLK, _N), lambda i: (i, 0)),
            pl.BlockSpec((1, _H, _QBLK, _D), lambda i, b=blk0: (0, 0, b + i, 0)),
            pl.BlockSpec((1, _H, _N, _D), lambda i: (0, 0, 0, 0)),
            pl.BlockSpec((1, _H, _N, _D), lambda i: (0, 0, 0, 0)),
        ],
        out_specs=pl.BlockSpec((1, _H, _QBLK, _D), lambda i: (0, 0, i, 0)),
        out_shape=jax.ShapeDtypeStruct((1, _H, nrows, _D), jnp.float32),
        compiler_params=pltpu.CompilerParams(
            dimension_semantics=("arbitrary",),
        ),
    )(bias, query, key, value)


@jax.jit
def kernel(query, key, value, attention_mask):
    mask2d = attention_mask[0]
    top = _TOP_ROWS
    bias_bot = _sc_topk_bias(mask2d, top, _N - top)
    out_top = _tc_fused(query, key, value, attention_mask, top)
    out_bot = _tc_attention(query, key, value, bias_bot, top, _N - top)
    return jnp.concatenate([out_top, out_bot], axis=2)
